# Initial kernel scaffold; baseline (speedup 1.0000x reference)
import jax
import jax.numpy as jnp
from jax.experimental import pallas as pl


def kernel(app_features, pose_features, mem_app, mem_pose, norm_app_min, norm_app_max, norm_pose_min, norm_pose_max):
    raise NotImplementedError("write your pallas kernel here")


# speedup vs baseline: 2.2129x; 2.2129x over previous
import functools

import jax
import jax.numpy as jnp
from jax.experimental import pallas as pl
from jax.experimental.pallas import tpu as pltpu

_Q = 1024
_M = 65536
_BLK = 1024
_STEPS = _M // _BLK


def _knn_body(appt_ref, poset_ref, ma_ref, mp_ref, oa_ref, op_ref,
              acc_a, acc_p):
    i = pl.program_id(0)

    @pl.when(i == 0)
    def _init():
        acc_a[...] = jnp.full((1, _Q), jnp.inf, jnp.float32)
        acc_p[...] = jnp.full((1, _Q), jnp.inf, jnp.float32)

    ma = ma_ref[...]
    ga = jnp.dot(ma, appt_ref[...],
                 preferred_element_type=jnp.float32)
    b2a = jnp.sum(ma * ma, axis=1, keepdims=True)
    ta = b2a - 2.0 * ga
    acc_a[...] = jnp.minimum(acc_a[...], jnp.min(ta, axis=0, keepdims=True))

    mp = mp_ref[...]
    gp = jnp.dot(mp, poset_ref[...],
                 preferred_element_type=jnp.float32)
    b2p = jnp.sum(mp * mp, axis=1, keepdims=True)
    tp = b2p - 2.0 * gp
    acc_p[...] = jnp.minimum(acc_p[...], jnp.min(tp, axis=0, keepdims=True))

    @pl.when(i == _STEPS - 1)
    def _fin():
        a2a = jnp.sum(appt_ref[...] * appt_ref[...], axis=0, keepdims=True)
        a2p = jnp.sum(poset_ref[...] * poset_ref[...], axis=0, keepdims=True)
        oa_ref[...] = jnp.sqrt(jnp.maximum(a2a + acc_a[...], 0.0))
        op_ref[...] = jnp.sqrt(jnp.maximum(a2p + acc_p[...], 0.0))


@functools.partial(jax.jit, static_argnames=())
def kernel(app_features, pose_features, mem_app, mem_pose,
           norm_app_min, norm_app_max, norm_pose_min, norm_pose_max):
    app_t = app_features.T
    pose_t = pose_features.T

    dist_a, dist_p = pl.pallas_call(
        _knn_body,
        grid=(_STEPS,),
        in_specs=[
            pl.BlockSpec((256, _Q), lambda i: (0, 0)),
            pl.BlockSpec((64, _Q), lambda i: (0, 0)),
            pl.BlockSpec((_BLK, 256), lambda i: (i, 0)),
            pl.BlockSpec((_BLK, 64), lambda i: (i, 0)),
        ],
        out_specs=[
            pl.BlockSpec((1, _Q), lambda i: (0, 0)),
            pl.BlockSpec((1, _Q), lambda i: (0, 0)),
        ],
        out_shape=[
            jax.ShapeDtypeStruct((1, _Q), jnp.float32),
            jax.ShapeDtypeStruct((1, _Q), jnp.float32),
        ],
        scratch_shapes=[
            pltpu.VMEM((1, _Q), jnp.float32),
            pltpu.VMEM((1, _Q), jnp.float32),
        ],
        compiler_params=pltpu.CompilerParams(
            dimension_semantics=("arbitrary",),
        ),
    )(app_t, pose_t, mem_app, mem_pose)

    score_a = (dist_a[0] - norm_app_min[0]) / (norm_app_max[0] - norm_app_min[0])
    score_p = (dist_p[0] - norm_pose_min[0]) / (norm_pose_max[0] - norm_pose_min[0])
    return score_a + score_p
